# Initial kernel scaffold; baseline (speedup 1.0000x reference)
import jax
import jax.numpy as jnp
from jax.experimental import pallas as pl


def kernel(x_user, x_item, edge_index_ui, edge_index_iu, Wu, bu, Wi, bi, Wr_rates_0, Wn_rates_0, b_rates_0, Wr_rev_0, Wn_rev_0, b_rev_0, Wr_rates_1, Wn_rates_1, b_rates_1, Wr_rev_1, Wn_rev_1, b_rev_1):
    raise NotImplementedError("write your pallas kernel here")


# speedup vs baseline: 10.7970x; 10.7970x over previous
import functools

import jax
import jax.numpy as jnp
from jax import lax
from jax.experimental import pallas as pl
from jax.experimental.pallas import tpu as pltpu
from jax.experimental.pallas import tpu_sc as plsc

N = 50000
E = 625000
H = 64
HH = 32
R = 50176
TPR = 3136
ITERS = 52
IDXROWS = 6
ROWS = 4992
EPAD = ROWS * 128


def _sc_mesh():
    return plsc.VectorSubcoreMesh(
        core_axis_name="c", subcore_axis_name="s", num_cores=2, num_subcores=16)


def _make_agg():
    out_type = [
        jax.ShapeDtypeStruct((2, R, HH), jnp.float32),
        jax.ShapeDtypeStruct((2, R, HH), jnp.float32),
    ]

    scratch = [
        pltpu.VMEM((IDXROWS, 128), jnp.int32),
        pltpu.VMEM((IDXROWS, 128), jnp.int32),
        pltpu.VMEM((IDXROWS, 128, HH), jnp.float32),
        pltpu.VMEM((64, HH), jnp.float32),
        pltpu.VMEM_SHARED((R, HH), jnp.float32),
        pltpu.SemaphoreType.DMA,
        pltpu.SemaphoreType.DMA,
    ]

    def body(tabU, tabI, src_ui, dst_ui, src_iu, dst_iu, zconst,
             sum_ui, sum_iu,
             sidx, didx, rows, zbuf, acc, gsem, ssem):
        c = lax.axis_index("c")
        s = lax.axis_index("s")

        pltpu.sync_copy(zconst, zbuf)

        def phase(tab, src_cat, dst_rows, sum_out):
            for k in range(TPR // 64):
                pltpu.sync_copy(zbuf, acc.at[pl.ds(s * TPR + k * 64, 64)])
            plsc.subcore_barrier()

            def it(w, carry):
                rb = s * (ITERS * IDXROWS) + w * IDXROWS
                pltpu.sync_copy(src_cat.at[pl.ds(c * ROWS + rb, IDXROWS)], sidx)
                pltpu.sync_copy(dst_rows.at[pl.ds(rb, IDXROWS)], didx)
                gd = [pltpu.async_copy(tab.at[sidx.at[j]], rows.at[j], gsem)
                      for j in range(IDXROWS)]
                for d in gd:
                    d.wait()
                sd = [pltpu.async_copy(rows.at[j], acc.at[didx.at[j]], ssem,
                                       add=True)
                      for j in range(IDXROWS)]
                for d in sd:
                    d.wait()
                return carry

            lax.fori_loop(0, ITERS, it, 0)
            plsc.subcore_barrier()
            pltpu.sync_copy(acc.at[pl.ds(s * TPR, TPR)],
                            sum_out.at[c, pl.ds(s * TPR, TPR)])

        phase(tabU, src_ui, dst_ui, sum_ui)
        phase(tabI, src_iu, dst_iu, sum_iu)

    return pl.kernel(body, out_type=out_type, mesh=_sc_mesh(),
                     scratch_types=scratch,
                     compiler_params=pltpu.CompilerParams(
                         use_tc_tiling_on_sc=False))


_agg = _make_agg()


def _count_body(dst_cat, oconst, z16const, cnt_out,
                didx, ones16, zbuf16, acc16, ssem):
    c = lax.axis_index("c")
    s = lax.axis_index("s")
    pltpu.sync_copy(oconst, ones16)
    pltpu.sync_copy(z16const, zbuf16)
    for k in range(TPR // 64):
        pltpu.sync_copy(zbuf16, acc16.at[pl.ds(s * TPR + k * 64, 64)])
    plsc.subcore_barrier()

    def it(w, carry):
        rb = c * ROWS + s * (ITERS * IDXROWS) + w * IDXROWS
        pltpu.sync_copy(dst_cat.at[pl.ds(rb, IDXROWS)], didx)
        cd = [pltpu.async_copy(ones16, acc16.at[didx.at[j]], ssem, add=True)
              for j in range(IDXROWS)]
        for d in cd:
            d.wait()
        return carry

    lax.fori_loop(0, ITERS, it, 0)
    plsc.subcore_barrier()
    pltpu.sync_copy(acc16.at[pl.ds(s * TPR, TPR)],
                    cnt_out.at[c, pl.ds(s * TPR, TPR)])


_count = pl.kernel(
    _count_body,
    out_type=jax.ShapeDtypeStruct((2, R, 16), jnp.float32),
    mesh=_sc_mesh(),
    scratch_types=[
        pltpu.VMEM((IDXROWS, 128), jnp.int32),
        pltpu.VMEM((128, 16), jnp.float32),
        pltpu.VMEM((64, 16), jnp.float32),
        pltpu.VMEM_SHARED((R, 16), jnp.float32),
        pltpu.SemaphoreType.DMA,
    ],
    compiler_params=pltpu.CompilerParams(use_tc_tiling_on_sc=False),
)


_BLK = 1000
_GRID = N // _BLK


def _full(shape):
    return pl.BlockSpec(shape, lambda i: tuple(0 for _ in shape))


def _rows(shape):
    return pl.BlockSpec(shape, lambda i: (i,) + tuple(0 for _ in shape[1:]))


def _halves(shape):
    return pl.BlockSpec(shape, lambda i: (0, i, 0))


def _t0_body(xu_ref, xi_ref, Wu_ref, bu_ref, Wi_ref, bi_ref, Wnr_ref, Wnv_ref,
             xou_ref, xoi_ref, hu_ref, hi_ref):
    xu = jnp.dot(xu_ref[...], Wu_ref[...],
                 preferred_element_type=jnp.float32) + bu_ref[...]
    xi = jnp.dot(xi_ref[...], Wi_ref[...],
                 preferred_element_type=jnp.float32) + bi_ref[...]
    xou_ref[...] = xu
    xoi_ref[...] = xi
    hu = jnp.dot(xu, Wnr_ref[...], preferred_element_type=jnp.float32)
    hi = jnp.dot(xi, Wnv_ref[...], preferred_element_type=jnp.float32)
    hu_ref[0] = hu[:, :HH]
    hu_ref[1] = hu[:, HH:]
    hi_ref[0] = hi[:, :HH]
    hi_ref[1] = hi[:, HH:]


_t0 = pl.pallas_call(
    _t0_body,
    grid=(_GRID,),
    in_specs=[_rows((_BLK, 32)), _rows((_BLK, 64)), _full((32, H)),
              _full((1, H)), _full((64, H)), _full((1, H)),
              _full((H, H)), _full((H, H))],
    out_specs=[_rows((_BLK, H)), _rows((_BLK, H)),
               _halves((2, _BLK, HH)), _halves((2, _BLK, HH))],
    out_shape=[jax.ShapeDtypeStruct((N, H), jnp.float32),
               jax.ShapeDtypeStruct((N, H), jnp.float32),
               jax.ShapeDtypeStruct((2, N, HH), jnp.float32),
               jax.ShapeDtypeStruct((2, N, HH), jnp.float32)],
)


def _combine(x_ref, W_ref, b_ref, sum_ref, cnt_ref):
    agg = jnp.concatenate([sum_ref[0], sum_ref[1]], axis=1)
    agg = agg / jnp.maximum(cnt_ref[:, 0:1], 1.0)
    out = jnp.dot(x_ref[...], W_ref[...],
                  preferred_element_type=jnp.float32) + agg + b_ref[...]
    return jnp.maximum(out, 0.0)


def _t1_body(xu_ref, xi_ref, si_ref, su_ref, cnt_ref, Wrr_ref, brr_ref,
             Wrv_ref, brv_ref, Wnr1_ref, Wnv1_ref,
             xu1_ref, xi1_ref, hu1_ref, hi1_ref):
    xi1 = _combine(xi_ref, Wrr_ref, brr_ref, si_ref, cnt_ref[0])
    xu1 = _combine(xu_ref, Wrv_ref, brv_ref, su_ref, cnt_ref[1])
    xu1_ref[...] = xu1
    xi1_ref[...] = xi1
    hu1 = jnp.dot(xu1, Wnr1_ref[...], preferred_element_type=jnp.float32)
    hi1 = jnp.dot(xi1, Wnv1_ref[...], preferred_element_type=jnp.float32)
    hu1_ref[0] = hu1[:, :HH]
    hu1_ref[1] = hu1[:, HH:]
    hi1_ref[0] = hi1[:, :HH]
    hi1_ref[1] = hi1[:, HH:]


_t1 = pl.pallas_call(
    _t1_body,
    grid=(_GRID,),
    in_specs=[_rows((_BLK, H)), _rows((_BLK, H)),
              _halves((2, _BLK, HH)), _halves((2, _BLK, HH)),
              _halves((2, _BLK, 16)),
              _full((H, H)), _full((1, H)), _full((H, H)), _full((1, H)),
              _full((H, H)), _full((H, H))],
    out_specs=[_rows((_BLK, H)), _rows((_BLK, H)),
               _halves((2, _BLK, HH)), _halves((2, _BLK, HH))],
    out_shape=[jax.ShapeDtypeStruct((N, H), jnp.float32),
               jax.ShapeDtypeStruct((N, H), jnp.float32),
               jax.ShapeDtypeStruct((2, N, HH), jnp.float32),
               jax.ShapeDtypeStruct((2, N, HH), jnp.float32)],
)


def _t2_body(xu_ref, xi_ref, si_ref, su_ref, cnt_ref, Wrr_ref, brr_ref,
             Wrv_ref, brv_ref, xu2_ref, xi2_ref):
    xi2_ref[...] = _combine(xi_ref, Wrr_ref, brr_ref, si_ref, cnt_ref[0])
    xu2_ref[...] = _combine(xu_ref, Wrv_ref, brv_ref, su_ref, cnt_ref[1])


_t2 = pl.pallas_call(
    _t2_body,
    grid=(_GRID,),
    in_specs=[_rows((_BLK, H)), _rows((_BLK, H)),
              _halves((2, _BLK, HH)), _halves((2, _BLK, HH)),
              _halves((2, _BLK, 16)),
              _full((H, H)), _full((1, H)), _full((H, H)), _full((1, H))],
    out_specs=[_rows((_BLK, H)), _rows((_BLK, H))],
    out_shape=[jax.ShapeDtypeStruct((N, H), jnp.float32),
               jax.ShapeDtypeStruct((N, H), jnp.float32)],
)


def kernel(x_user, x_item, edge_index_ui, edge_index_iu, Wu, bu, Wi, bi,
           Wr_rates_0, Wn_rates_0, b_rates_0, Wr_rev_0, Wn_rev_0, b_rev_0,
           Wr_rates_1, Wn_rates_1, b_rates_1, Wr_rev_1, Wn_rev_1, b_rev_1):
    pad = EPAD - E
    ar = jnp.arange(pad, dtype=jnp.int32)
    pad_src = ar % 4096
    pad_dst = N + (ar % 128)

    def prep(edge):
        src = jnp.concatenate([edge[0], pad_src])
        dst = jnp.concatenate([edge[1], pad_dst])
        src_cat = jnp.concatenate([src, src + N]).reshape(2 * ROWS, 128)
        return src_cat, dst.reshape(ROWS, 128)

    src_ui, dst_ui = prep(edge_index_ui)
    src_iu, dst_iu = prep(edge_index_iu)

    zconst = jnp.zeros((64, HH), jnp.float32)
    oconst = jnp.ones((128, 16), jnp.float32)
    z16const = jnp.zeros((64, 16), jnp.float32)
    dst_cat = jnp.concatenate([dst_ui, dst_iu], axis=0)

    bu2 = bu.reshape(1, H)
    bi2 = bi.reshape(1, H)

    x_u, x_i, hu0, hi0 = _t0(x_user, x_item, Wu, bu2, Wi, bi2,
                             Wn_rates_0, Wn_rev_0)

    cnt = _count(dst_cat, oconst, z16const)

    sum_i0, sum_u0 = _agg(
        hu0.reshape(2 * N, HH), hi0.reshape(2 * N, HH),
        src_ui, dst_ui, src_iu, dst_iu, zconst)

    x_u1, x_i1, hu1, hi1 = _t1(
        x_u, x_i, sum_i0, sum_u0, cnt,
        Wr_rates_0, b_rates_0.reshape(1, H), Wr_rev_0, b_rev_0.reshape(1, H),
        Wn_rates_1, Wn_rev_1)

    sum_i1, sum_u1 = _agg(
        hu1.reshape(2 * N, HH), hi1.reshape(2 * N, HH),
        src_ui, dst_ui, src_iu, dst_iu, zconst)

    x_u2, x_i2 = _t2(
        x_u1, x_i1, sum_i1, sum_u1, cnt,
        Wr_rates_1, b_rates_1.reshape(1, H), Wr_rev_1, b_rev_1.reshape(1, H))

    return (x_u2, x_i2)
